# baseline (speedup 1.0000x reference)
import jax
import jax.numpy as jnp
from jax import lax
from jax.experimental import pallas as pl
from jax.experimental.pallas import tpu as pltpu
from jax.experimental.pallas import tpu_sc as plsc

BATCH = 16384
NCARDS = 7
DIM = 128
NCHUNK = DIM // 16

NC = 2
NS = 16
NW = NC * NS
BPW = BATCH // NW
CROWS = 128
NCHUNKS = BPW // CROWS
GPC = CROWS // 16


def _sc_body(x_hbm, card_hbm, rank_hbm, suit_hbm, out_hbm,
             card_v, rank_v, suit_v, t_v, x_v, o0_v, o1_v,
             sx, so0, so1):
    wid = lax.axis_index("s") * NC + lax.axis_index("c")
    row0 = wid * BPW
    obufs = (o0_v, o1_v)
    osems = (so0, so1)

    xd = pltpu.async_copy(x_hbm.at[:, pl.ds(row0, BPW)], x_v, sx)

    pltpu.sync_copy(card_hbm, card_v)
    pltpu.sync_copy(rank_hbm, rank_v)
    pltpu.sync_copy(suit_hbm, suit_v)

    @plsc.parallel_loop(1, 52)
    def _build(i):
        q = i // 4
        m = i - q * 4
        for j in range(NCHUNK):
            js = pl.ds(j * 16, 16)
            t_v[pl.ds(i * DIM + j * 16, 16)] = (
                card_v[i, js] + rank_v[q, js] + suit_v[m, js])

    zero = jnp.zeros((16,), jnp.float32)
    for j in range(NCHUNK):
        t_v[pl.ds(j * 16, 16)] = zero

    xd.wait()

    od = [None] * NCHUNKS
    for k in range(NCHUNKS):
        ob = obufs[k % 2]
        if k >= 2:
            od[k - 2].wait()

        @plsc.parallel_loop(0, GPC)
        def _group(g):
            gcol = k * CROWS + g * 16
            idx = [x_v[c, pl.ds(gcol, 16)] for c in range(NCARDS)]
            for r in range(16):
                base = [idx[c][r] * DIM for c in range(NCARDS)]
                orow = g * 16 + r
                for j in range(NCHUNK):
                    js = j * 16
                    t0 = t_v[pl.ds(base[0] + js, 16)]
                    t1 = t_v[pl.ds(base[1] + js, 16)]
                    t2 = t_v[pl.ds(base[2] + js, 16)]
                    t3 = t_v[pl.ds(base[3] + js, 16)]
                    t4 = t_v[pl.ds(base[4] + js, 16)]
                    t5 = t_v[pl.ds(base[5] + js, 16)]
                    t6 = t_v[pl.ds(base[6] + js, 16)]
                    ob[orow, pl.ds(js, 16)] = (
                        ((t0 + t1) + (t2 + t3)) + ((t4 + t5) + t6))

        od[k] = pltpu.async_copy(
            ob, out_hbm.at[pl.ds(row0 + k * CROWS, CROWS)], osems[k % 2])
    od[NCHUNKS - 2].wait()
    od[NCHUNKS - 1].wait()


@jax.jit
def kernel(X, card, rank, suit):
    xt = X.astype(jnp.int32).T
    f = pl.kernel(
        _sc_body,
        out_type=jax.ShapeDtypeStruct((BATCH, DIM), jnp.float32),
        mesh=plsc.VectorSubcoreMesh(core_axis_name="c", subcore_axis_name="s"),
        compiler_params=pltpu.CompilerParams(needs_layout_passes=False),
        scratch_types=[
            pltpu.VMEM((52, DIM), jnp.float32),
            pltpu.VMEM((13, DIM), jnp.float32),
            pltpu.VMEM((4, DIM), jnp.float32),
            pltpu.VMEM((52 * DIM,), jnp.float32),
            pltpu.VMEM((NCARDS, BPW), jnp.int32),
            pltpu.VMEM((CROWS, DIM), jnp.float32),
            pltpu.VMEM((CROWS, DIM), jnp.float32),
            pltpu.SemaphoreType.DMA,
            pltpu.SemaphoreType.DMA,
            pltpu.SemaphoreType.DMA,
        ],
    )
    return f(xt, card.astype(jnp.float32),
             rank.astype(jnp.float32), suit.astype(jnp.float32))


# speedup vs baseline: 2.7921x; 2.7921x over previous
import jax
import jax.numpy as jnp
from jax import lax
from jax.experimental import pallas as pl
from jax.experimental.pallas import tpu as pltpu
from jax.experimental.pallas import tpu_sc as plsc

BATCH = 16384
NCARDS = 7
DIM = 128
NCHUNK = DIM // 16

NC = 2
NS = 16
NW = NC * NS
BPW = BATCH // NW
CROWS = 128
NCHUNKS = BPW // CROWS
GPC = CROWS // 16


def _sc_body(x_hbm, card_hbm, rank_hbm, suit_hbm, out_hbm,
             card_v, rank_v, suit_v, t_v, x_v, xf_v, o0_v, o1_v,
             sx, so0, so1):
    wid = lax.axis_index("s") * NC + lax.axis_index("c")
    row0 = wid * BPW
    obufs = (o0_v, o1_v)
    osems = (so0, so1)

    xd = pltpu.async_copy(x_hbm.at[:, pl.ds(row0, BPW)], x_v, sx)

    pltpu.sync_copy(card_hbm, card_v)
    pltpu.sync_copy(rank_hbm, rank_v)
    pltpu.sync_copy(suit_hbm, suit_v)

    @plsc.parallel_loop(1, 52)
    def _build(i):
        q = i // 4
        m = i - q * 4
        for j in range(NCHUNK):
            js = pl.ds(j * 16, 16)
            t_v[pl.ds(i * DIM + j * 16, 16)] = (
                card_v[i, js] + rank_v[q, js] + suit_v[m, js])

    zero = jnp.zeros((16,), jnp.float32)
    for j in range(NCHUNK):
        t_v[pl.ds(j * 16, 16)] = zero

    xd.wait()

    lane = lax.iota(jnp.int32, 16)
    lane7 = lane * NCARDS

    @plsc.parallel_loop(0, BPW // 16)
    def _xpose(g):
        gbase = g * (16 * NCARDS)
        for c in range(NCARDS):
            plsc.store_scatter(xf_v, [lane7 + (gbase + c)],
                               x_v[c, pl.ds(g * 16, 16)])

    od = [None] * NCHUNKS
    for k in range(NCHUNKS):
        ob = obufs[k % 2]
        if k >= 2:
            od[k - 2].wait()

        @plsc.parallel_loop(0, CROWS, unroll=2)
        def _row(b):
            xrow = xf_v[pl.ds((k * CROWS + b) * NCARDS, 16)]
            base = [xrow[c] * DIM for c in range(NCARDS)]
            for j in range(NCHUNK):
                js = j * 16
                t0 = t_v[pl.ds(base[0] + js, 16)]
                t1 = t_v[pl.ds(base[1] + js, 16)]
                t2 = t_v[pl.ds(base[2] + js, 16)]
                t3 = t_v[pl.ds(base[3] + js, 16)]
                t4 = t_v[pl.ds(base[4] + js, 16)]
                t5 = t_v[pl.ds(base[5] + js, 16)]
                t6 = t_v[pl.ds(base[6] + js, 16)]
                ob[b, pl.ds(js, 16)] = (
                    ((t0 + t1) + (t2 + t3)) + ((t4 + t5) + t6))

        od[k] = pltpu.async_copy(
            ob, out_hbm.at[pl.ds(row0 + k * CROWS, CROWS)], osems[k % 2])
    od[NCHUNKS - 2].wait()
    od[NCHUNKS - 1].wait()


@jax.jit
def kernel(X, card, rank, suit):
    xt = X.astype(jnp.int32).T
    f = pl.kernel(
        _sc_body,
        out_type=jax.ShapeDtypeStruct((BATCH, DIM), jnp.float32),
        mesh=plsc.VectorSubcoreMesh(core_axis_name="c", subcore_axis_name="s"),
        compiler_params=pltpu.CompilerParams(needs_layout_passes=False),
        scratch_types=[
            pltpu.VMEM((52, DIM), jnp.float32),
            pltpu.VMEM((13, DIM), jnp.float32),
            pltpu.VMEM((4, DIM), jnp.float32),
            pltpu.VMEM((52 * DIM,), jnp.float32),
            pltpu.VMEM((NCARDS, BPW), jnp.int32),
            pltpu.VMEM((BPW * NCARDS + 16,), jnp.int32),
            pltpu.VMEM((CROWS, DIM), jnp.float32),
            pltpu.VMEM((CROWS, DIM), jnp.float32),
            pltpu.SemaphoreType.DMA,
            pltpu.SemaphoreType.DMA,
            pltpu.SemaphoreType.DMA,
        ],
    )
    return f(xt, card.astype(jnp.float32),
             rank.astype(jnp.float32), suit.astype(jnp.float32))


# speedup vs baseline: 3.2964x; 1.1806x over previous
import jax
import jax.numpy as jnp
from jax import lax
from jax.experimental import pallas as pl
from jax.experimental.pallas import tpu as pltpu
from jax.experimental.pallas import tpu_sc as plsc

BATCH = 16384
NCARDS = 7
DIM = 128
NCHUNK = DIM // 16

NC = 2
NS = 16
NW = NC * NS
BPW = BATCH // NW
CROWS = 128
NCHUNKS = BPW // CROWS
GPC = CROWS // 16


def _sc_body(x_hbm, card_hbm, rank_hbm, suit_hbm, out_hbm,
             card_v, rank_v, suit_v, t_v, x_v, xf_v, o0_v, o1_v,
             sx, so0, so1):
    wid = lax.axis_index("s") * NC + lax.axis_index("c")
    row0 = wid * BPW
    obufs = (o0_v, o1_v)
    osems = (so0, so1)

    xd = pltpu.async_copy(x_hbm.at[:, pl.ds(row0, BPW)], x_v, sx)

    pltpu.sync_copy(card_hbm, card_v)
    pltpu.sync_copy(rank_hbm, rank_v)
    pltpu.sync_copy(suit_hbm, suit_v)

    @plsc.parallel_loop(1, 52)
    def _build(i):
        q = i // 4
        m = i - q * 4
        for j in range(NCHUNK // 2):
            ja = pl.ds(j * 32, 16)
            jb = pl.ds(j * 32 + 16, 16)
            a = card_v[i, ja] + rank_v[q, ja] + suit_v[m, ja]
            b = card_v[i, jb] + rank_v[q, jb] + suit_v[m, jb]
            t_v[pl.ds(i * DIM + j * 32, 32)] = plsc.pack(
                a, b, format=plsc.PackFormat.INTERLEAVED)

    zero = jnp.zeros((32,), jnp.bfloat16)
    for j in range(NCHUNK // 2):
        t_v[pl.ds(j * 32, 32)] = zero

    xd.wait()

    lane = lax.iota(jnp.int32, 16)
    lane7 = lane * NCARDS

    @plsc.parallel_loop(0, BPW // 16)
    def _xpose(g):
        gbase = g * (16 * NCARDS)
        for c in range(NCARDS):
            plsc.store_scatter(xf_v, [lane7 + (gbase + c)],
                               x_v[c, pl.ds(g * 16, 16)])

    od = [None] * NCHUNKS
    for k in range(NCHUNKS):
        ob = obufs[k % 2]
        if k >= 2:
            od[k - 2].wait()

        @plsc.parallel_loop(0, CROWS, unroll=2)
        def _row(b):
            xrow = xf_v[pl.ds((k * CROWS + b) * NCARDS, 16)]
            base = [xrow[c] * DIM for c in range(NCARDS)]
            for j in range(NCHUNK // 2):
                js = j * 32
                ts = [plsc.unpack(t_v[pl.ds(base[c] + js, 32)],
                                  format=plsc.PackFormat.INTERLEAVED)
                      for c in range(NCARDS)]
                acc_a = (((ts[0][0] + ts[1][0]) + (ts[2][0] + ts[3][0]))
                         + ((ts[4][0] + ts[5][0]) + ts[6][0]))
                acc_b = (((ts[0][1] + ts[1][1]) + (ts[2][1] + ts[3][1]))
                         + ((ts[4][1] + ts[5][1]) + ts[6][1]))
                ob[b, pl.ds(js, 16)] = acc_a
                ob[b, pl.ds(js + 16, 16)] = acc_b

        od[k] = pltpu.async_copy(
            ob, out_hbm.at[pl.ds(row0 + k * CROWS, CROWS)], osems[k % 2])
    od[NCHUNKS - 2].wait()
    od[NCHUNKS - 1].wait()


@jax.jit
def kernel(X, card, rank, suit):
    xt = X.astype(jnp.int32).T
    f = pl.kernel(
        _sc_body,
        out_type=jax.ShapeDtypeStruct((BATCH, DIM), jnp.float32),
        mesh=plsc.VectorSubcoreMesh(core_axis_name="c", subcore_axis_name="s"),
        compiler_params=pltpu.CompilerParams(needs_layout_passes=False),
        scratch_types=[
            pltpu.VMEM((52, DIM), jnp.float32),
            pltpu.VMEM((13, DIM), jnp.float32),
            pltpu.VMEM((4, DIM), jnp.float32),
            pltpu.VMEM((52 * DIM,), jnp.bfloat16),
            pltpu.VMEM((NCARDS, BPW), jnp.int32),
            pltpu.VMEM((BPW * NCARDS + 16,), jnp.int32),
            pltpu.VMEM((CROWS, DIM), jnp.float32),
            pltpu.VMEM((CROWS, DIM), jnp.float32),
            pltpu.SemaphoreType.DMA,
            pltpu.SemaphoreType.DMA,
            pltpu.SemaphoreType.DMA,
        ],
    )
    return f(xt, card.astype(jnp.float32),
             rank.astype(jnp.float32), suit.astype(jnp.float32))


# speedup vs baseline: 3.3053x; 1.0027x over previous
import jax
import jax.numpy as jnp
from jax import lax
from jax.experimental import pallas as pl
from jax.experimental.pallas import tpu as pltpu
from jax.experimental.pallas import tpu_sc as plsc

BATCH = 16384
NCARDS = 7
DIM = 128
NCHUNK = DIM // 16

NC = 2
NS = 16
NW = NC * NS
BPW = BATCH // NW
CROWS = 128
NCHUNKS = BPW // CROWS
GPC = CROWS // 16


def _sc_body(x_hbm, card_hbm, rank_hbm, suit_hbm, out_hbm,
             card_v, rank_v, suit_v, t_v, x_v, xf_v, o0_v, o1_v,
             sx, so0, so1):
    wid = lax.axis_index("s") * NC + lax.axis_index("c")
    row0 = wid * BPW
    obufs = (o0_v, o1_v)
    osems = (so0, so1)

    xd = pltpu.async_copy(x_hbm.at[:, pl.ds(row0, BPW)], x_v, sx)

    pltpu.sync_copy(card_hbm, card_v)
    pltpu.sync_copy(rank_hbm, rank_v)
    pltpu.sync_copy(suit_hbm, suit_v)

    rnd = jnp.full((16,), 0x8000, jnp.int32)
    himask = jnp.full((16,), -65536, jnp.int32)

    @plsc.parallel_loop(1, 52)
    def _build(i):
        q = i // 4
        m = i - q * 4
        for j in range(NCHUNK // 2):
            ja = pl.ds(j * 32, 16)
            jb = pl.ds(j * 32 + 16, 16)
            a = card_v[i, ja] + rank_v[q, ja] + suit_v[m, ja]
            b = card_v[i, jb] + rank_v[q, jb] + suit_v[m, jb]
            a32 = lax.bitcast_convert_type(a, jnp.int32) + rnd
            b32 = lax.bitcast_convert_type(b, jnp.int32) + rnd
            w = lax.shift_right_logical(a32, jnp.full((16,), 16, jnp.int32)) | (b32 & himask)
            t_v[pl.ds(i * (DIM // 2) + j * 16, 16)] = w

    zero = jnp.zeros((16,), jnp.int32)
    for j in range(NCHUNK // 2):
        t_v[pl.ds(j * 16, 16)] = zero

    xd.wait()

    lane = lax.iota(jnp.int32, 16)
    lane7 = lane * NCARDS

    @plsc.parallel_loop(0, BPW // 16)
    def _xpose(g):
        gbase = g * (16 * NCARDS)
        for c in range(NCARDS):
            plsc.store_scatter(xf_v, [lane7 + (gbase + c)],
                               x_v[c, pl.ds(g * 16, 16)])

    od = [None] * NCHUNKS
    for k in range(NCHUNKS):
        ob = obufs[k % 2]
        if k >= 2:
            od[k - 2].wait()

        @plsc.parallel_loop(0, CROWS, unroll=2)
        def _row(b):
            xrow = xf_v[pl.ds((k * CROWS + b) * NCARDS, 16)]
            base = [xrow[c] * (DIM // 2) for c in range(NCARDS)]
            sh16 = jnp.full((16,), 16, jnp.int32)
            for j in range(NCHUNK // 2):
                ws = [t_v[pl.ds(base[c] + j * 16, 16)] for c in range(NCARDS)]
                ta = [lax.bitcast_convert_type(
                          lax.shift_left(w, sh16), jnp.float32) for w in ws]
                tb = [lax.bitcast_convert_type(w & himask, jnp.float32)
                      for w in ws]
                acc_a = (((ta[0] + ta[1]) + (ta[2] + ta[3]))
                         + ((ta[4] + ta[5]) + ta[6]))
                acc_b = (((tb[0] + tb[1]) + (tb[2] + tb[3]))
                         + ((tb[4] + tb[5]) + tb[6]))
                ob[b, pl.ds(j * 32, 16)] = acc_a
                ob[b, pl.ds(j * 32 + 16, 16)] = acc_b

        od[k] = pltpu.async_copy(
            ob, out_hbm.at[pl.ds(row0 + k * CROWS, CROWS)], osems[k % 2])
    od[NCHUNKS - 2].wait()
    od[NCHUNKS - 1].wait()


@jax.jit
def kernel(X, card, rank, suit):
    xt = X.astype(jnp.int32).T
    f = pl.kernel(
        _sc_body,
        out_type=jax.ShapeDtypeStruct((BATCH, DIM), jnp.float32),
        mesh=plsc.VectorSubcoreMesh(core_axis_name="c", subcore_axis_name="s"),
        compiler_params=pltpu.CompilerParams(needs_layout_passes=False),
        scratch_types=[
            pltpu.VMEM((52, DIM), jnp.float32),
            pltpu.VMEM((13, DIM), jnp.float32),
            pltpu.VMEM((4, DIM), jnp.float32),
            pltpu.VMEM((52 * (DIM // 2),), jnp.int32),
            pltpu.VMEM((NCARDS, BPW), jnp.int32),
            pltpu.VMEM((BPW * NCARDS + 16,), jnp.int32),
            pltpu.VMEM((CROWS, DIM), jnp.float32),
            pltpu.VMEM((CROWS, DIM), jnp.float32),
            pltpu.SemaphoreType.DMA,
            pltpu.SemaphoreType.DMA,
            pltpu.SemaphoreType.DMA,
        ],
    )
    return f(xt, card.astype(jnp.float32),
             rank.astype(jnp.float32), suit.astype(jnp.float32))


# speedup vs baseline: 3.3119x; 1.0020x over previous
import jax
import jax.numpy as jnp
from jax import lax
from jax.experimental import pallas as pl
from jax.experimental.pallas import tpu as pltpu
from jax.experimental.pallas import tpu_sc as plsc

BATCH = 16384
NCARDS = 7
DIM = 128
NCHUNK = DIM // 16

NC = 2
NS = 16
NW = NC * NS
BPW = BATCH // NW
CROWS = 128
NCHUNKS = BPW // CROWS


def _sc_body(x_hbm, card_hbm, rank_hbm, suit_hbm, out_hbm,
             card_v, rank_v, suit_v, t_v, x_v, xf_v, o0_v, o1_v,
             sx, so0, so1):
    wid = lax.axis_index("s") * NC + lax.axis_index("c")
    row0 = wid * BPW
    obufs = (o0_v, o1_v)
    osems = (so0, so1)

    xd = pltpu.async_copy(x_hbm.at[:, pl.ds(row0, BPW)], x_v, sx)

    pltpu.sync_copy(card_hbm, card_v)
    pltpu.sync_copy(rank_hbm, rank_v)
    pltpu.sync_copy(suit_hbm, suit_v)

    rnd = jnp.full((16,), 0x8000, jnp.int32)
    himask = jnp.full((16,), -65536, jnp.int32)

    @plsc.parallel_loop(1, 52)
    def _build(i):
        q = i // 4
        m = i - q * 4
        for j in range(NCHUNK // 2):
            ja = pl.ds(j * 32, 16)
            jb = pl.ds(j * 32 + 16, 16)
            a = card_v[i, ja] + rank_v[q, ja] + suit_v[m, ja]
            b = card_v[i, jb] + rank_v[q, jb] + suit_v[m, jb]
            a32 = lax.bitcast_convert_type(a, jnp.int32) + rnd
            b32 = lax.bitcast_convert_type(b, jnp.int32) + rnd
            w = lax.shift_right_logical(a32, jnp.full((16,), 16, jnp.int32)) | (b32 & himask)
            t_v[pl.ds(i * (DIM // 2) + j * 16, 16)] = w

    zero = jnp.zeros((16,), jnp.int32)
    for j in range(NCHUNK // 2):
        t_v[pl.ds(j * 16, 16)] = zero

    xd.wait()

    lane = lax.iota(jnp.int32, 16)
    lane7 = lane * NCARDS

    @plsc.parallel_loop(0, BPW // 16)
    def _xpose(g):
        gbase = g * (16 * NCARDS)
        for c in range(NCARDS):
            plsc.store_scatter(xf_v, [lane7 + (gbase + c)],
                               x_v[c, pl.ds(g * 16, 16)])

    od = [None] * NCHUNKS
    for k in range(NCHUNKS):
        ob = obufs[k % 2]
        if k >= 2:
            od[k - 2].wait()

        @plsc.parallel_loop(0, CROWS, unroll=2)
        def _row(b):
            xrow = xf_v[pl.ds((k * CROWS + b) * NCARDS, 16)]
            base = [xrow[c] * (DIM // 2) for c in range(NCARDS)]
            sh16 = jnp.full((16,), 16, jnp.int32)
            for j in range(NCHUNK // 2):
                ws = [t_v[pl.ds(base[c] + j * 16, 16)] for c in range(NCARDS)]
                ta = [lax.bitcast_convert_type(
                          lax.shift_left(w, sh16), jnp.float32) for w in ws]
                tb = [lax.bitcast_convert_type(w & himask, jnp.float32)
                      for w in ws]
                acc_a = (((ta[0] + ta[1]) + (ta[2] + ta[3]))
                         + ((ta[4] + ta[5]) + ta[6]))
                acc_b = (((tb[0] + tb[1]) + (tb[2] + tb[3]))
                         + ((tb[4] + tb[5]) + tb[6]))
                ob[b, pl.ds(j * 32, 16)] = acc_a
                ob[b, pl.ds(j * 32 + 16, 16)] = acc_b

        od[k] = pltpu.async_copy(
            ob, out_hbm.at[pl.ds(row0 + k * CROWS, CROWS)], osems[k % 2])
    od[NCHUNKS - 2].wait()
    od[NCHUNKS - 1].wait()


@jax.jit
def kernel(X, card, rank, suit):
    xt = X.astype(jnp.int32).T
    f = pl.kernel(
        _sc_body,
        out_type=jax.ShapeDtypeStruct((BATCH, DIM), jnp.float32),
        mesh=plsc.VectorSubcoreMesh(core_axis_name="c", subcore_axis_name="s"),
        compiler_params=pltpu.CompilerParams(needs_layout_passes=False),
        scratch_types=[
            pltpu.VMEM((52, DIM), jnp.float32),
            pltpu.VMEM((13, DIM), jnp.float32),
            pltpu.VMEM((4, DIM), jnp.float32),
            pltpu.VMEM((52 * (DIM // 2),), jnp.int32),
            pltpu.VMEM((NCARDS, BPW), jnp.int32),
            pltpu.VMEM((BPW * NCARDS + 16,), jnp.int32),
            pltpu.VMEM((CROWS, DIM), jnp.float32),
            pltpu.VMEM((CROWS, DIM), jnp.float32),
            pltpu.SemaphoreType.DMA,
            pltpu.SemaphoreType.DMA,
            pltpu.SemaphoreType.DMA,
        ],
    )
    return f(xt, card.astype(jnp.float32),
             rank.astype(jnp.float32), suit.astype(jnp.float32))
